# baseline (speedup 1.0000x reference)
import functools

import jax
import jax.numpy as jnp
from jax import lax
from jax.experimental import pallas as pl
from jax.experimental.pallas import tpu as pltpu
from jax.experimental.pallas import tpu_sc as plsc

N_SEGMENTS = 1_000_000
T_FINAL = 1.0
N_CHANNELS = 16
N_TIMES = 3_276_800

NUM_CORES = 2
NUM_SUBCORES = 16
LANES = 16
NUM_WORKERS = NUM_CORES * NUM_SUBCORES
B_PER_WORKER = N_TIMES // NUM_WORKERS

CHUNK = 1024
N_CHUNKS = B_PER_WORKER // CHUNK
GATHER_W = 128
KG = CHUNK // GATHER_W
QB = CHUNK // 128
N_QB = N_TIMES // 128
SCALE = float(N_SEGMENTS / T_FINAL)


def _sc_gather(t_hbm, amp_hbm, out_hbm,
               t_v0, t_v1, idx_v0, idx_v1, rows0, rows1, ob0, ob1,
               semt0, semt1, semg0, semg1, semw0, semw1):
    wid = lax.axis_index("s") * NUM_CORES + lax.axis_index("c")
    base = wid * B_PER_WORKER
    qb_base = wid * (B_PER_WORKER // 128)

    def start_t(g, t_v, semt):
        pltpu.async_copy(t_hbm.at[pl.ds(base + g * CHUNK, CHUNK)], t_v, semt)

    def wait_t(g, t_v, semt):
        pltpu.make_async_copy(
            t_hbm.at[pl.ds(base + g * CHUNK, CHUNK)], t_v, semt).wait()

    def compute_idx(t_v, idx_v):
        def body(r, carry):
            for c in range(GATHER_W // LANES):
                tv = t_v[pl.ds(r * GATHER_W + c * LANES, LANES)]
                ix = (tv * SCALE).astype(jnp.int32)
                ix = jnp.minimum(jnp.maximum(ix, 0), N_SEGMENTS - 1)
                idx_v[r, pl.ds(c * LANES, LANES)] = ix
            return carry

        lax.fori_loop(0, KG, body, 0, unroll=False)

    def fire_gathers(idx_v, rows_v, semg):
        for j in range(KG):
            pltpu.async_copy(
                amp_hbm.at[idx_v.at[j]],
                rows_v.at[pl.ds(j * GATHER_W, GATHER_W)], semg)

    def drain_gathers(idx_v, rows_v, semg):
        for j in range(KG):
            pltpu.make_async_copy(
                amp_hbm.at[idx_v.at[j]],
                rows_v.at[pl.ds(j * GATHER_W, GATHER_W)], semg).wait()

    HALF = QB * 1024
    iot = lax.iota(jnp.int32, LANES)
    pat0 = (iot >> 3) * HALF + (iot & 7) * 128

    def transpose(rows_v, obuf):
        def body(k, carry):
            q0 = k * LANES
            dynbase = ((q0 >> 7) << 10) + (q0 & 127)
            for u in range(LANES):
                v = rows_v[q0 + u]
                plsc.store_scatter(obuf, [pat0 + (dynbase + u)], v)
            return carry

        lax.fori_loop(0, CHUNK // LANES, body, 0, unroll=False)

    def start_wb(g, obuf, semw):
        qo = (qb_base + g * QB) * 1024
        pltpu.async_copy(obuf.at[pl.ds(0, HALF)],
                         out_hbm.at[0, pl.ds(qo, HALF)], semw)
        pltpu.async_copy(obuf.at[pl.ds(HALF, HALF)],
                         out_hbm.at[1, pl.ds(qo, HALF)], semw)

    def wait_wb(g, obuf, semw):
        qo = (qb_base + g * QB) * 1024
        pltpu.make_async_copy(obuf.at[pl.ds(0, HALF)],
                              out_hbm.at[0, pl.ds(qo, HALF)], semw).wait()
        pltpu.make_async_copy(obuf.at[pl.ds(HALF, HALF)],
                              out_hbm.at[1, pl.ds(qo, HALF)], semw).wait()

    slot = [(t_v0, idx_v0, rows0, ob0, semt0, semg0, semw0),
            (t_v1, idx_v1, rows1, ob1, semt1, semg1, semw1)]

    start_t(0, t_v0, semt0)
    start_t(1, t_v1, semt1)
    wait_t(0, t_v0, semt0)
    compute_idx(t_v0, idx_v0)
    start_t(2, t_v0, semt0)
    fire_gathers(idx_v0, rows0, semg0)
    wait_t(1, t_v1, semt1)
    compute_idx(t_v1, idx_v1)
    start_t(3, t_v1, semt1)
    fire_gathers(idx_v1, rows1, semg1)
    drain_gathers(idx_v0, rows0, semg0)
    transpose(rows0, ob0)
    start_wb(0, ob0, semw0)
    wait_t(2, t_v0, semt0)
    compute_idx(t_v0, idx_v0)
    start_t(4, t_v0, semt0)
    fire_gathers(idx_v0, rows0, semg0)
    drain_gathers(idx_v1, rows1, semg1)
    transpose(rows1, ob1)
    start_wb(1, ob1, semw1)
    wait_t(3, t_v1, semt1)
    compute_idx(t_v1, idx_v1)
    start_t(5, t_v1, semt1)
    fire_gathers(idx_v1, rows1, semg1)
    drain_gathers(idx_v0, rows0, semg0)
    wait_wb(0, ob0, semw0)
    transpose(rows0, ob0)
    start_wb(2, ob0, semw0)

    def step(s, carry):
        g0 = 2 * s
        g1 = g0 + 1
        wait_t(g0, t_v0, semt0)
        compute_idx(t_v0, idx_v0)
        start_t(g0 + 2, t_v0, semt0)
        fire_gathers(idx_v0, rows0, semg0)
        drain_gathers(idx_v1, rows1, semg1)
        wait_wb(g0 - 3, ob1, semw1)
        transpose(rows1, ob1)
        start_wb(g0 - 1, ob1, semw1)
        wait_t(g1, t_v1, semt1)
        compute_idx(t_v1, idx_v1)
        start_t(g1 + 2, t_v1, semt1)
        fire_gathers(idx_v1, rows1, semg1)
        drain_gathers(idx_v0, rows0, semg0)
        wait_wb(g1 - 3, ob0, semw0)
        transpose(rows0, ob0)
        start_wb(g0, ob0, semw0)
        return carry

    lax.fori_loop(2, N_CHUNKS // 2 - 1, step, 0, unroll=False)

    gA = N_CHUNKS - 2
    gB = N_CHUNKS - 1
    wait_t(gA, t_v0, semt0)
    compute_idx(t_v0, idx_v0)
    fire_gathers(idx_v0, rows0, semg0)
    drain_gathers(idx_v1, rows1, semg1)
    wait_wb(gA - 3, ob1, semw1)
    transpose(rows1, ob1)
    start_wb(gA - 1, ob1, semw1)

    wait_t(gB, t_v1, semt1)
    compute_idx(t_v1, idx_v1)
    fire_gathers(idx_v1, rows1, semg1)
    drain_gathers(idx_v0, rows0, semg0)
    wait_wb(gB - 3, ob0, semw0)
    transpose(rows0, ob0)
    start_wb(gA, ob0, semw0)

    drain_gathers(idx_v1, rows1, semg1)
    wait_wb(gA - 1, ob1, semw1)
    transpose(rows1, ob1)
    start_wb(gB, ob1, semw1)
    wait_wb(gA, ob0, semw0)
    wait_wb(gB, ob1, semw1)


@jax.jit
def kernel(t, amplitudes):
    mesh = plsc.VectorSubcoreMesh(core_axis_name="c", subcore_axis_name="s")
    run = functools.partial(
        pl.kernel,
        mesh=mesh,
        out_type=jax.ShapeDtypeStruct((2, N_QB * 1024), jnp.float32),
        scratch_types=[
            pltpu.VMEM((CHUNK,), jnp.float32),
            pltpu.VMEM((CHUNK,), jnp.float32),
            pltpu.VMEM((KG, GATHER_W), jnp.int32),
            pltpu.VMEM((KG, GATHER_W), jnp.int32),
            pltpu.VMEM((CHUNK, N_CHANNELS), jnp.float32),
            pltpu.VMEM((CHUNK, N_CHANNELS), jnp.float32),
            pltpu.VMEM((2 * QB * 1024,), jnp.float32),
            pltpu.VMEM((2 * QB * 1024,), jnp.float32),
            pltpu.SemaphoreType.DMA,
            pltpu.SemaphoreType.DMA,
            pltpu.SemaphoreType.DMA,
            pltpu.SemaphoreType.DMA,
            pltpu.SemaphoreType.DMA,
            pltpu.SemaphoreType.DMA,
        ],
        compiler_params=pltpu.CompilerParams(
            use_tc_tiling_on_sc=False, needs_layout_passes=False),
    )(_sc_gather)
    out4 = run(t, amplitudes).reshape(2, N_QB, 8, 128)
    return out4.transpose(1, 3, 0, 2).reshape(N_TIMES, N_CHANNELS)


# speedup vs baseline: 1.1905x; 1.1905x over previous
import functools

import jax
import jax.numpy as jnp
from jax import lax
from jax.experimental import pallas as pl
from jax.experimental.pallas import tpu as pltpu
from jax.experimental.pallas import tpu_sc as plsc

N_SEGMENTS = 1_000_000
T_FINAL = 1.0
N_CHANNELS = 16
N_TIMES = 3_276_800

NUM_CORES = 2
NUM_SUBCORES = 16
LANES = 16
NUM_WORKERS = NUM_CORES * NUM_SUBCORES
B_PER_WORKER = N_TIMES // NUM_WORKERS

CHUNK = 1024
N_CHUNKS = B_PER_WORKER // CHUNK
GATHER_W = 128
KG = CHUNK // GATHER_W
QB = CHUNK // 128
N_QB = N_TIMES // 128
SCALE = float(N_SEGMENTS / T_FINAL)


def _sc_gather(t_hbm, amp_hbm, out_hbm,
               t_v0, t_v1, idx_v0, idx_v1, rows0, rows1, ob0, ob1,
               semt0, semt1, semg0, semg1, semw0, semw1):
    wid = lax.axis_index("s") * NUM_CORES + lax.axis_index("c")
    base = wid * B_PER_WORKER
    qb_base = wid * (B_PER_WORKER // 128)

    def start_t(g, t_v, semt):
        pltpu.async_copy(t_hbm.at[pl.ds(base + g * CHUNK, CHUNK)], t_v, semt)

    def wait_t(g, t_v, semt):
        pltpu.make_async_copy(
            t_hbm.at[pl.ds(base + g * CHUNK, CHUNK)], t_v, semt).wait()

    def compute_idx(t_v, idx_v):
        def body(r, carry):
            for c in range(GATHER_W // LANES):
                tv = t_v[pl.ds(r * GATHER_W + c * LANES, LANES)]
                ix = (tv * SCALE).astype(jnp.int32)
                ix = jnp.minimum(jnp.maximum(ix, 0), N_SEGMENTS - 1)
                idx_v[r, pl.ds(c * LANES, LANES)] = ix
            return carry

        lax.fori_loop(0, KG, body, 0, unroll=False)

    def fire_gathers(idx_v, rows_v, semg):
        for j in range(KG):
            pltpu.async_copy(
                amp_hbm.at[idx_v.at[j]],
                rows_v.at[pl.ds(j * GATHER_W, GATHER_W)], semg)

    def drain_gathers(idx_v, rows_v, semg):
        for j in range(KG):
            pltpu.make_async_copy(
                amp_hbm.at[idx_v.at[j]],
                rows_v.at[pl.ds(j * GATHER_W, GATHER_W)], semg).wait()

    HALF = QB * 1024
    iot = lax.iota(jnp.int32, LANES)
    pat0 = (iot >> 3) * HALF + (iot & 7) * 128

    def transpose(rows_v, obuf):
        def body(k, carry):
            q0 = k * LANES
            dynbase = ((q0 >> 7) << 10) + (q0 & 127)
            vs = [rows_v[q0 + u] for u in range(LANES)]
            idxs = [pat0 + (dynbase + u) for u in range(LANES)]
            for u in range(LANES):
                plsc.store_scatter(obuf, [idxs[u]], vs[u])
            return carry

        lax.fori_loop(0, CHUNK // LANES, body, 0, unroll=False)

    def start_wb(g, obuf, semw):
        qo = (qb_base + g * QB) * 1024
        pltpu.async_copy(obuf.at[pl.ds(0, HALF)],
                         out_hbm.at[0, pl.ds(qo, HALF)], semw)
        pltpu.async_copy(obuf.at[pl.ds(HALF, HALF)],
                         out_hbm.at[1, pl.ds(qo, HALF)], semw)

    def wait_wb(g, obuf, semw):
        qo = (qb_base + g * QB) * 1024
        pltpu.make_async_copy(obuf.at[pl.ds(0, HALF)],
                              out_hbm.at[0, pl.ds(qo, HALF)], semw).wait()
        pltpu.make_async_copy(obuf.at[pl.ds(HALF, HALF)],
                              out_hbm.at[1, pl.ds(qo, HALF)], semw).wait()

    slot = [(t_v0, idx_v0, rows0, ob0, semt0, semg0, semw0),
            (t_v1, idx_v1, rows1, ob1, semt1, semg1, semw1)]

    start_t(0, t_v0, semt0)
    start_t(1, t_v1, semt1)
    wait_t(0, t_v0, semt0)
    compute_idx(t_v0, idx_v0)
    start_t(2, t_v0, semt0)
    fire_gathers(idx_v0, rows0, semg0)
    wait_t(1, t_v1, semt1)
    compute_idx(t_v1, idx_v1)
    start_t(3, t_v1, semt1)
    fire_gathers(idx_v1, rows1, semg1)
    drain_gathers(idx_v0, rows0, semg0)
    transpose(rows0, ob0)
    start_wb(0, ob0, semw0)
    wait_t(2, t_v0, semt0)
    compute_idx(t_v0, idx_v0)
    start_t(4, t_v0, semt0)
    fire_gathers(idx_v0, rows0, semg0)
    drain_gathers(idx_v1, rows1, semg1)
    transpose(rows1, ob1)
    start_wb(1, ob1, semw1)
    wait_t(3, t_v1, semt1)
    compute_idx(t_v1, idx_v1)
    start_t(5, t_v1, semt1)
    fire_gathers(idx_v1, rows1, semg1)
    drain_gathers(idx_v0, rows0, semg0)
    wait_wb(0, ob0, semw0)
    transpose(rows0, ob0)
    start_wb(2, ob0, semw0)

    def step(s, carry):
        g0 = 2 * s
        g1 = g0 + 1
        wait_t(g0, t_v0, semt0)
        compute_idx(t_v0, idx_v0)
        start_t(g0 + 2, t_v0, semt0)
        fire_gathers(idx_v0, rows0, semg0)
        drain_gathers(idx_v1, rows1, semg1)
        wait_wb(g0 - 3, ob1, semw1)
        transpose(rows1, ob1)
        start_wb(g0 - 1, ob1, semw1)
        wait_t(g1, t_v1, semt1)
        compute_idx(t_v1, idx_v1)
        start_t(g1 + 2, t_v1, semt1)
        fire_gathers(idx_v1, rows1, semg1)
        drain_gathers(idx_v0, rows0, semg0)
        wait_wb(g1 - 3, ob0, semw0)
        transpose(rows0, ob0)
        start_wb(g0, ob0, semw0)
        return carry

    lax.fori_loop(2, N_CHUNKS // 2 - 1, step, 0, unroll=False)

    gA = N_CHUNKS - 2
    gB = N_CHUNKS - 1
    wait_t(gA, t_v0, semt0)
    compute_idx(t_v0, idx_v0)
    fire_gathers(idx_v0, rows0, semg0)
    drain_gathers(idx_v1, rows1, semg1)
    wait_wb(gA - 3, ob1, semw1)
    transpose(rows1, ob1)
    start_wb(gA - 1, ob1, semw1)

    wait_t(gB, t_v1, semt1)
    compute_idx(t_v1, idx_v1)
    fire_gathers(idx_v1, rows1, semg1)
    drain_gathers(idx_v0, rows0, semg0)
    wait_wb(gB - 3, ob0, semw0)
    transpose(rows0, ob0)
    start_wb(gA, ob0, semw0)

    drain_gathers(idx_v1, rows1, semg1)
    wait_wb(gA - 1, ob1, semw1)
    transpose(rows1, ob1)
    start_wb(gB, ob1, semw1)
    wait_wb(gA, ob0, semw0)
    wait_wb(gB, ob1, semw1)


@jax.jit
def kernel(t, amplitudes):
    mesh = plsc.VectorSubcoreMesh(core_axis_name="c", subcore_axis_name="s")
    run = functools.partial(
        pl.kernel,
        mesh=mesh,
        out_type=jax.ShapeDtypeStruct((2, N_QB * 1024), jnp.float32),
        scratch_types=[
            pltpu.VMEM((CHUNK,), jnp.float32),
            pltpu.VMEM((CHUNK,), jnp.float32),
            pltpu.VMEM((KG, GATHER_W), jnp.int32),
            pltpu.VMEM((KG, GATHER_W), jnp.int32),
            pltpu.VMEM((CHUNK, N_CHANNELS), jnp.float32),
            pltpu.VMEM((CHUNK, N_CHANNELS), jnp.float32),
            pltpu.VMEM((2 * QB * 1024,), jnp.float32),
            pltpu.VMEM((2 * QB * 1024,), jnp.float32),
            pltpu.SemaphoreType.DMA,
            pltpu.SemaphoreType.DMA,
            pltpu.SemaphoreType.DMA,
            pltpu.SemaphoreType.DMA,
            pltpu.SemaphoreType.DMA,
            pltpu.SemaphoreType.DMA,
        ],
        compiler_params=pltpu.CompilerParams(
            use_tc_tiling_on_sc=False, needs_layout_passes=False,
            disable_bounds_checks=True),
    )(_sc_gather)
    out4 = run(t, amplitudes).reshape(2, N_QB, 8, 128)
    return out4.transpose(1, 3, 0, 2).reshape(N_TIMES, N_CHANNELS)


# speedup vs baseline: 2.0559x; 1.7269x over previous
import functools

import jax
import jax.numpy as jnp
from jax import lax
from jax.experimental import pallas as pl
from jax.experimental.pallas import tpu as pltpu
from jax.experimental.pallas import tpu_sc as plsc

N_SEGMENTS = 1_000_000
T_FINAL = 1.0
N_CHANNELS = 16
N_TIMES = 3_276_800

NUM_CORES = 2
NUM_SUBCORES = 16
LANES = 16
NUM_WORKERS = NUM_CORES * NUM_SUBCORES
B_PER_WORKER = N_TIMES // NUM_WORKERS

CHUNK = 1024
N_CHUNKS = B_PER_WORKER // CHUNK
GATHER_W = 128
KG = CHUNK // GATHER_W
QB = CHUNK // 128
N_QB = N_TIMES // 128
SCALE = float(N_SEGMENTS / T_FINAL)


def _sc_gather(t_hbm, amp_hbm, out_hbm,
               t_v0, t_v1, idx_v0, idx_v1, rows0, rows1, ob0, ob1, sbuf,
               semt0, semt1, semg0, semg1, semw0, semw1):
    wid = lax.axis_index("s") * NUM_CORES + lax.axis_index("c")
    base = wid * B_PER_WORKER
    qb_base = wid * (B_PER_WORKER // 128)

    def start_t(g, t_v, semt):
        pltpu.async_copy(t_hbm.at[pl.ds(base + g * CHUNK, CHUNK)], t_v, semt)

    def wait_t(g, t_v, semt):
        pltpu.make_async_copy(
            t_hbm.at[pl.ds(base + g * CHUNK, CHUNK)], t_v, semt).wait()

    def compute_idx(t_v, idx_v):
        def body(r, carry):
            for c in range(GATHER_W // LANES):
                tv = t_v[pl.ds(r * GATHER_W + c * LANES, LANES)]
                ix = (tv * SCALE).astype(jnp.int32)
                ix = jnp.minimum(jnp.maximum(ix, 0), N_SEGMENTS - 1)
                idx_v[r, pl.ds(c * LANES, LANES)] = ix
            return carry

        lax.fori_loop(0, KG, body, 0, unroll=False)

    def fire_gathers(idx_v, rows_v, semg):
        for j in range(KG):
            pltpu.async_copy(
                amp_hbm.at[idx_v.at[j]],
                rows_v.at[pl.ds(j * GATHER_W, GATHER_W)], semg)

    def drain_gathers(idx_v, rows_v, semg):
        for j in range(KG):
            pltpu.make_async_copy(
                amp_hbm.at[idx_v.at[j]],
                rows_v.at[pl.ds(j * GATHER_W, GATHER_W)], semg).wait()

    HALF = QB * 1024
    iot = lax.iota(jnp.int32, LANES)
    base17 = iot * 17

    def transpose(rows_v, obuf, sbuf):
        def body(k, carry):
            q0 = k * LANES
            dynbase = ((q0 >> 7) << 10) + (q0 & 127)
            vs = [rows_v[q0 + u] for u in range(LANES)]
            for u in range(LANES):
                sbuf[pl.ds(u * 17, LANES)] = vs[u]
            cols = [plsc.load_gather(sbuf, [base17 + c]) for c in range(16)]
            for c in range(16):
                off_c = (c // 8) * HALF + (c % 8) * 128
                obuf[pl.ds(dynbase + off_c, LANES)] = cols[c]
            return carry

        lax.fori_loop(0, CHUNK // LANES, body, 0, unroll=False)

    def start_wb(g, obuf, semw):
        qo = (qb_base + g * QB) * 1024
        pltpu.async_copy(obuf.at[pl.ds(0, HALF)],
                         out_hbm.at[0, pl.ds(qo, HALF)], semw)
        pltpu.async_copy(obuf.at[pl.ds(HALF, HALF)],
                         out_hbm.at[1, pl.ds(qo, HALF)], semw)

    def wait_wb(g, obuf, semw):
        qo = (qb_base + g * QB) * 1024
        pltpu.make_async_copy(obuf.at[pl.ds(0, HALF)],
                              out_hbm.at[0, pl.ds(qo, HALF)], semw).wait()
        pltpu.make_async_copy(obuf.at[pl.ds(HALF, HALF)],
                              out_hbm.at[1, pl.ds(qo, HALF)], semw).wait()

    slot = [(t_v0, idx_v0, rows0, ob0, semt0, semg0, semw0),
            (t_v1, idx_v1, rows1, ob1, semt1, semg1, semw1)]

    start_t(0, t_v0, semt0)
    start_t(1, t_v1, semt1)
    wait_t(0, t_v0, semt0)
    compute_idx(t_v0, idx_v0)
    start_t(2, t_v0, semt0)
    fire_gathers(idx_v0, rows0, semg0)
    wait_t(1, t_v1, semt1)
    compute_idx(t_v1, idx_v1)
    start_t(3, t_v1, semt1)
    fire_gathers(idx_v1, rows1, semg1)
    drain_gathers(idx_v0, rows0, semg0)
    transpose(rows0, ob0, sbuf)
    start_wb(0, ob0, semw0)
    wait_t(2, t_v0, semt0)
    compute_idx(t_v0, idx_v0)
    start_t(4, t_v0, semt0)
    fire_gathers(idx_v0, rows0, semg0)
    drain_gathers(idx_v1, rows1, semg1)
    transpose(rows1, ob1, sbuf)
    start_wb(1, ob1, semw1)
    wait_t(3, t_v1, semt1)
    compute_idx(t_v1, idx_v1)
    start_t(5, t_v1, semt1)
    fire_gathers(idx_v1, rows1, semg1)
    drain_gathers(idx_v0, rows0, semg0)
    wait_wb(0, ob0, semw0)
    transpose(rows0, ob0, sbuf)
    start_wb(2, ob0, semw0)

    def step(s, carry):
        g0 = 2 * s
        g1 = g0 + 1
        wait_t(g0, t_v0, semt0)
        compute_idx(t_v0, idx_v0)
        start_t(g0 + 2, t_v0, semt0)
        fire_gathers(idx_v0, rows0, semg0)
        drain_gathers(idx_v1, rows1, semg1)
        wait_wb(g0 - 3, ob1, semw1)
        transpose(rows1, ob1, sbuf)
        start_wb(g0 - 1, ob1, semw1)
        wait_t(g1, t_v1, semt1)
        compute_idx(t_v1, idx_v1)
        start_t(g1 + 2, t_v1, semt1)
        fire_gathers(idx_v1, rows1, semg1)
        drain_gathers(idx_v0, rows0, semg0)
        wait_wb(g1 - 3, ob0, semw0)
        transpose(rows0, ob0, sbuf)
        start_wb(g0, ob0, semw0)
        return carry

    lax.fori_loop(2, N_CHUNKS // 2 - 1, step, 0, unroll=False)

    gA = N_CHUNKS - 2
    gB = N_CHUNKS - 1
    wait_t(gA, t_v0, semt0)
    compute_idx(t_v0, idx_v0)
    fire_gathers(idx_v0, rows0, semg0)
    drain_gathers(idx_v1, rows1, semg1)
    wait_wb(gA - 3, ob1, semw1)
    transpose(rows1, ob1, sbuf)
    start_wb(gA - 1, ob1, semw1)

    wait_t(gB, t_v1, semt1)
    compute_idx(t_v1, idx_v1)
    fire_gathers(idx_v1, rows1, semg1)
    drain_gathers(idx_v0, rows0, semg0)
    wait_wb(gB - 3, ob0, semw0)
    transpose(rows0, ob0, sbuf)
    start_wb(gA, ob0, semw0)

    drain_gathers(idx_v1, rows1, semg1)
    wait_wb(gA - 1, ob1, semw1)
    transpose(rows1, ob1, sbuf)
    start_wb(gB, ob1, semw1)
    wait_wb(gA, ob0, semw0)
    wait_wb(gB, ob1, semw1)


@jax.jit
def kernel(t, amplitudes):
    mesh = plsc.VectorSubcoreMesh(core_axis_name="c", subcore_axis_name="s")
    run = functools.partial(
        pl.kernel,
        mesh=mesh,
        out_type=jax.ShapeDtypeStruct((2, N_QB * 1024), jnp.float32),
        scratch_types=[
            pltpu.VMEM((CHUNK,), jnp.float32),
            pltpu.VMEM((CHUNK,), jnp.float32),
            pltpu.VMEM((KG, GATHER_W), jnp.int32),
            pltpu.VMEM((KG, GATHER_W), jnp.int32),
            pltpu.VMEM((CHUNK, N_CHANNELS), jnp.float32),
            pltpu.VMEM((CHUNK, N_CHANNELS), jnp.float32),
            pltpu.VMEM((2 * QB * 1024,), jnp.float32),
            pltpu.VMEM((2 * QB * 1024,), jnp.float32),
            pltpu.VMEM((16 * 17,), jnp.float32),
            pltpu.SemaphoreType.DMA,
            pltpu.SemaphoreType.DMA,
            pltpu.SemaphoreType.DMA,
            pltpu.SemaphoreType.DMA,
            pltpu.SemaphoreType.DMA,
            pltpu.SemaphoreType.DMA,
        ],
        compiler_params=pltpu.CompilerParams(
            use_tc_tiling_on_sc=False, needs_layout_passes=False,
            disable_bounds_checks=True),
    )(_sc_gather)
    out4 = run(t, amplitudes).reshape(2, N_QB, 8, 128)
    return out4.transpose(1, 3, 0, 2).reshape(N_TIMES, N_CHANNELS)


# speedup vs baseline: 3.2418x; 1.5769x over previous
import functools

import jax
import jax.numpy as jnp
from jax import lax
from jax.experimental import pallas as pl
from jax.experimental.pallas import tpu as pltpu
from jax.experimental.pallas import tpu_sc as plsc

N_SEGMENTS = 1_000_000
T_FINAL = 1.0
N_CHANNELS = 16
N_TIMES = 3_276_800

NUM_CORES = 2
NUM_SUBCORES = 16
LANES = 16
NUM_WORKERS = NUM_CORES * NUM_SUBCORES
B_PER_WORKER = N_TIMES // NUM_WORKERS

CHUNK = 1024
N_CHUNKS = B_PER_WORKER // CHUNK
GATHER_W = 128
KG = CHUNK // GATHER_W
QB = CHUNK // 128
N_QB = N_TIMES // 128
SCALE = float(N_SEGMENTS / T_FINAL)


N_BLK = 7813
N_ROWS_PAD = N_BLK * 16
ITERS_A = 245


def _sc_format(ampT_hbm, outA_hbm, ib0, ib1, oba0, oba1, sbufA,
               semi0, semi1, semo0, semo1):
    wid = lax.axis_index("s") * NUM_CORES + lax.axis_index("c")
    iota = lax.iota(jnp.int32, LANES)
    b17 = iota * 17

    def bclamp(i):
        return jnp.minimum(i * NUM_WORKERS + wid, N_BLK - 1)

    def start_in(i, ib, semi):
        b = bclamp(i)
        pltpu.async_copy(
            ampT_hbm.at[pl.ds(0, 16), pl.ds(128 * b, 128)], ib, semi)

    def wait_in(i, ib, semi):
        b = bclamp(i)
        pltpu.make_async_copy(
            ampT_hbm.at[pl.ds(0, 16), pl.ds(128 * b, 128)], ib, semi).wait()

    def start_out(i, oba, semo):
        b = bclamp(i)
        pltpu.async_copy(oba, outA_hbm.at[pl.ds(16 * b, 16)], semo)

    def wait_out(i, oba, semo):
        b = bclamp(i)
        pltpu.make_async_copy(
            oba, outA_hbm.at[pl.ds(16 * b, 16)], semo).wait()

    def transblock(ib, oba):
        def body(m, carry):
            for u in range(16):
                sbufA[pl.ds(u * 17, LANES)] = ib[u, pl.ds(m * LANES, LANES)]
            cols = [plsc.load_gather(sbufA, [b17 + c]) for c in range(16)]
            for c in range(16):
                oba[2 * m + (c // 8), pl.ds((c % 8) * LANES, LANES)] = cols[c]
            return carry

        lax.fori_loop(0, 8, body, 0, unroll=False)

    def process(i, ib, oba, semi, semo, first):
        wait_in(i, ib, semi)
        if not first:
            wait_out(i - 2, oba, semo)
        transblock(ib, oba)
        start_in(i + 2, ib, semi)
        start_out(i, oba, semo)

    start_in(0, ib0, semi0)
    start_in(1, ib1, semi1)
    process(0, ib0, oba0, semi0, semo0, True)
    process(1, ib1, oba1, semi1, semo1, True)
    process(2, ib0, oba0, semi0, semo0, False)

    def stepA(p, carry):
        i0 = 3 + 2 * p
        process(i0, ib1, oba1, semi1, semo1, False)
        process(i0 + 1, ib0, oba0, semi0, semo0, False)
        return carry

    lax.fori_loop(0, (ITERS_A - 3) // 2, stepA, 0, unroll=False)

    wait_in(ITERS_A, ib1, semi1)
    wait_in(ITERS_A + 1, ib0, semi0)
    wait_out(ITERS_A - 2, oba1, semo1)
    wait_out(ITERS_A - 1, oba0, semo0)


def _sc_gather(t_hbm, amp_hbm, out_hbm,
               t_v0, t_v1, idx_v0, idx_v1, rows0, rows1, ob0, ob1, sbuf,
               semt0, semt1, semg0, semg1, semw0, semw1):
    wid = lax.axis_index("s") * NUM_CORES + lax.axis_index("c")
    base = wid * B_PER_WORKER
    qb_base = wid * (B_PER_WORKER // 128)

    def start_t(g, t_v, semt):
        pltpu.async_copy(t_hbm.at[pl.ds(base + g * CHUNK, CHUNK)], t_v, semt)

    def wait_t(g, t_v, semt):
        pltpu.make_async_copy(
            t_hbm.at[pl.ds(base + g * CHUNK, CHUNK)], t_v, semt).wait()

    def compute_idx(t_v, idx_v):
        def body(r, carry):
            for c in range(GATHER_W // LANES):
                tv = t_v[pl.ds(r * GATHER_W + c * LANES, LANES)]
                ix = (tv * SCALE).astype(jnp.int32)
                ix = jnp.minimum(jnp.maximum(ix, 0), N_SEGMENTS - 1)
                idx_v[r, pl.ds(c * LANES, LANES)] = ix
            return carry

        lax.fori_loop(0, KG, body, 0, unroll=False)

    def fire_gathers(idx_v, rows_v, semg):
        for j in range(KG):
            pltpu.async_copy(
                amp_hbm.at[idx_v.at[j]],
                rows_v.at[pl.ds(j * GATHER_W, GATHER_W)], semg)

    def drain_gathers(idx_v, rows_v, semg):
        for j in range(KG):
            pltpu.make_async_copy(
                amp_hbm.at[idx_v.at[j]],
                rows_v.at[pl.ds(j * GATHER_W, GATHER_W)], semg).wait()

    HALF = QB * 1024
    iot = lax.iota(jnp.int32, LANES)
    base17 = iot * 17

    def transpose(rows_v, obuf, sbuf):
        def body(k, carry):
            q0 = k * LANES
            dynbase = ((q0 >> 7) << 10) + (q0 & 127)
            vs = [rows_v[q0 + u] for u in range(LANES)]
            for u in range(LANES):
                sbuf[pl.ds(u * 17, LANES)] = vs[u]
            cols = [plsc.load_gather(sbuf, [base17 + c]) for c in range(16)]
            for c in range(16):
                off_c = (c // 8) * HALF + (c % 8) * 128
                obuf[pl.ds(dynbase + off_c, LANES)] = cols[c]
            return carry

        lax.fori_loop(0, CHUNK // LANES, body, 0, unroll=False)

    def start_wb(g, obuf, semw):
        qo = (qb_base + g * QB) * 1024
        pltpu.async_copy(obuf.at[pl.ds(0, HALF)],
                         out_hbm.at[0, pl.ds(qo, HALF)], semw)
        pltpu.async_copy(obuf.at[pl.ds(HALF, HALF)],
                         out_hbm.at[1, pl.ds(qo, HALF)], semw)

    def wait_wb(g, obuf, semw):
        qo = (qb_base + g * QB) * 1024
        pltpu.make_async_copy(obuf.at[pl.ds(0, HALF)],
                              out_hbm.at[0, pl.ds(qo, HALF)], semw).wait()
        pltpu.make_async_copy(obuf.at[pl.ds(HALF, HALF)],
                              out_hbm.at[1, pl.ds(qo, HALF)], semw).wait()

    slot = [(t_v0, idx_v0, rows0, ob0, semt0, semg0, semw0),
            (t_v1, idx_v1, rows1, ob1, semt1, semg1, semw1)]

    start_t(0, t_v0, semt0)
    start_t(1, t_v1, semt1)
    wait_t(0, t_v0, semt0)
    compute_idx(t_v0, idx_v0)
    start_t(2, t_v0, semt0)
    fire_gathers(idx_v0, rows0, semg0)
    wait_t(1, t_v1, semt1)
    compute_idx(t_v1, idx_v1)
    start_t(3, t_v1, semt1)
    fire_gathers(idx_v1, rows1, semg1)
    drain_gathers(idx_v0, rows0, semg0)
    transpose(rows0, ob0, sbuf)
    start_wb(0, ob0, semw0)
    wait_t(2, t_v0, semt0)
    compute_idx(t_v0, idx_v0)
    start_t(4, t_v0, semt0)
    fire_gathers(idx_v0, rows0, semg0)
    drain_gathers(idx_v1, rows1, semg1)
    transpose(rows1, ob1, sbuf)
    start_wb(1, ob1, semw1)
    wait_t(3, t_v1, semt1)
    compute_idx(t_v1, idx_v1)
    start_t(5, t_v1, semt1)
    fire_gathers(idx_v1, rows1, semg1)
    drain_gathers(idx_v0, rows0, semg0)
    wait_wb(0, ob0, semw0)
    transpose(rows0, ob0, sbuf)
    start_wb(2, ob0, semw0)

    def step(s, carry):
        g0 = 2 * s
        g1 = g0 + 1
        wait_t(g0, t_v0, semt0)
        compute_idx(t_v0, idx_v0)
        start_t(g0 + 2, t_v0, semt0)
        fire_gathers(idx_v0, rows0, semg0)
        drain_gathers(idx_v1, rows1, semg1)
        wait_wb(g0 - 3, ob1, semw1)
        transpose(rows1, ob1, sbuf)
        start_wb(g0 - 1, ob1, semw1)
        wait_t(g1, t_v1, semt1)
        compute_idx(t_v1, idx_v1)
        start_t(g1 + 2, t_v1, semt1)
        fire_gathers(idx_v1, rows1, semg1)
        drain_gathers(idx_v0, rows0, semg0)
        wait_wb(g1 - 3, ob0, semw0)
        transpose(rows0, ob0, sbuf)
        start_wb(g0, ob0, semw0)
        return carry

    lax.fori_loop(2, N_CHUNKS // 2 - 1, step, 0, unroll=False)

    gA = N_CHUNKS - 2
    gB = N_CHUNKS - 1
    wait_t(gA, t_v0, semt0)
    compute_idx(t_v0, idx_v0)
    fire_gathers(idx_v0, rows0, semg0)
    drain_gathers(idx_v1, rows1, semg1)
    wait_wb(gA - 3, ob1, semw1)
    transpose(rows1, ob1, sbuf)
    start_wb(gA - 1, ob1, semw1)

    wait_t(gB, t_v1, semt1)
    compute_idx(t_v1, idx_v1)
    fire_gathers(idx_v1, rows1, semg1)
    drain_gathers(idx_v0, rows0, semg0)
    wait_wb(gB - 3, ob0, semw0)
    transpose(rows0, ob0, sbuf)
    start_wb(gA, ob0, semw0)

    drain_gathers(idx_v1, rows1, semg1)
    wait_wb(gA - 1, ob1, semw1)
    transpose(rows1, ob1, sbuf)
    start_wb(gB, ob1, semw1)
    wait_wb(gA, ob0, semw0)
    wait_wb(gB, ob1, semw1)


@jax.jit
def kernel(t, amplitudes):
    mesh = plsc.VectorSubcoreMesh(core_axis_name="c", subcore_axis_name="s")
    fmt = functools.partial(
        pl.kernel,
        mesh=mesh,
        out_type=jax.ShapeDtypeStruct((N_ROWS_PAD, 128), jnp.float32),
        scratch_types=[
            pltpu.VMEM((16, 128), jnp.float32),
            pltpu.VMEM((16, 128), jnp.float32),
            pltpu.VMEM((16, 128), jnp.float32),
            pltpu.VMEM((16, 128), jnp.float32),
            pltpu.VMEM((16 * 17,), jnp.float32),
            pltpu.SemaphoreType.DMA,
            pltpu.SemaphoreType.DMA,
            pltpu.SemaphoreType.DMA,
            pltpu.SemaphoreType.DMA,
        ],
        compiler_params=pltpu.CompilerParams(
            use_tc_tiling_on_sc=True, needs_layout_passes=False,
            disable_bounds_checks=True),
    )(_sc_format)
    amp_lin = fmt(amplitudes.T).reshape(N_ROWS_PAD * 8, 16)
    run = functools.partial(
        pl.kernel,
        mesh=mesh,
        out_type=jax.ShapeDtypeStruct((2, N_QB * 1024), jnp.float32),
        scratch_types=[
            pltpu.VMEM((CHUNK,), jnp.float32),
            pltpu.VMEM((CHUNK,), jnp.float32),
            pltpu.VMEM((KG, GATHER_W), jnp.int32),
            pltpu.VMEM((KG, GATHER_W), jnp.int32),
            pltpu.VMEM((CHUNK, N_CHANNELS), jnp.float32),
            pltpu.VMEM((CHUNK, N_CHANNELS), jnp.float32),
            pltpu.VMEM((2 * QB * 1024,), jnp.float32),
            pltpu.VMEM((2 * QB * 1024,), jnp.float32),
            pltpu.VMEM((16 * 17,), jnp.float32),
            pltpu.SemaphoreType.DMA,
            pltpu.SemaphoreType.DMA,
            pltpu.SemaphoreType.DMA,
            pltpu.SemaphoreType.DMA,
            pltpu.SemaphoreType.DMA,
            pltpu.SemaphoreType.DMA,
        ],
        compiler_params=pltpu.CompilerParams(
            use_tc_tiling_on_sc=False, needs_layout_passes=False,
            disable_bounds_checks=True),
    )(_sc_gather)
    out4 = run(t, amp_lin).reshape(2, N_QB, 8, 128)
    return out4.transpose(1, 3, 0, 2).reshape(N_TIMES, N_CHANNELS)


# speedup vs baseline: 3.2583x; 1.0051x over previous
import functools

import jax
import jax.numpy as jnp
from jax import lax
from jax.experimental import pallas as pl
from jax.experimental.pallas import tpu as pltpu
from jax.experimental.pallas import tpu_sc as plsc

N_SEGMENTS = 1_000_000
T_FINAL = 1.0
N_CHANNELS = 16
N_TIMES = 3_276_800

NUM_CORES = 2
NUM_SUBCORES = 16
LANES = 16
NUM_WORKERS = NUM_CORES * NUM_SUBCORES
B_PER_WORKER = N_TIMES // NUM_WORKERS

CHUNK = 1280
N_CHUNKS = B_PER_WORKER // CHUNK
GATHER_W = 128
KG = CHUNK // GATHER_W
QB = CHUNK // 128
N_QB = N_TIMES // 128
SCALE = float(N_SEGMENTS / T_FINAL)


N_BLK = 7813
N_ROWS_PAD = N_BLK * 16
ITERS_A = 245


def _sc_format(ampT_hbm, outA_hbm, ib0, ib1, oba0, oba1, sbufA,
               semi0, semi1, semo0, semo1):
    wid = lax.axis_index("s") * NUM_CORES + lax.axis_index("c")
    iota = lax.iota(jnp.int32, LANES)
    b17 = iota * 17

    def bclamp(i):
        return jnp.minimum(i * NUM_WORKERS + wid, N_BLK - 1)

    def start_in(i, ib, semi):
        b = bclamp(i)
        pltpu.async_copy(
            ampT_hbm.at[pl.ds(0, 16), pl.ds(128 * b, 128)], ib, semi)

    def wait_in(i, ib, semi):
        b = bclamp(i)
        pltpu.make_async_copy(
            ampT_hbm.at[pl.ds(0, 16), pl.ds(128 * b, 128)], ib, semi).wait()

    def start_out(i, oba, semo):
        b = bclamp(i)
        pltpu.async_copy(oba, outA_hbm.at[pl.ds(16 * b, 16)], semo)

    def wait_out(i, oba, semo):
        b = bclamp(i)
        pltpu.make_async_copy(
            oba, outA_hbm.at[pl.ds(16 * b, 16)], semo).wait()

    def transblock(ib, oba):
        def body(m, carry):
            for u in range(16):
                sbufA[pl.ds(u * 17, LANES)] = ib[u, pl.ds(m * LANES, LANES)]
            cols = [plsc.load_gather(sbufA, [b17 + c]) for c in range(16)]
            for c in range(16):
                oba[2 * m + (c // 8), pl.ds((c % 8) * LANES, LANES)] = cols[c]
            return carry

        lax.fori_loop(0, 8, body, 0, unroll=False)

    def process(i, ib, oba, semi, semo, first):
        wait_in(i, ib, semi)
        if not first:
            wait_out(i - 2, oba, semo)
        transblock(ib, oba)
        start_in(i + 2, ib, semi)
        start_out(i, oba, semo)

    start_in(0, ib0, semi0)
    start_in(1, ib1, semi1)
    process(0, ib0, oba0, semi0, semo0, True)
    process(1, ib1, oba1, semi1, semo1, True)
    process(2, ib0, oba0, semi0, semo0, False)

    def stepA(p, carry):
        i0 = 3 + 2 * p
        process(i0, ib1, oba1, semi1, semo1, False)
        process(i0 + 1, ib0, oba0, semi0, semo0, False)
        return carry

    lax.fori_loop(0, (ITERS_A - 3) // 2, stepA, 0, unroll=False)

    wait_in(ITERS_A, ib1, semi1)
    wait_in(ITERS_A + 1, ib0, semi0)
    wait_out(ITERS_A - 2, oba1, semo1)
    wait_out(ITERS_A - 1, oba0, semo0)


def _sc_gather(t_hbm, amp_hbm, out_hbm,
               t_v0, t_v1, idx_v0, idx_v1, rows0, rows1, ob0, ob1, sbuf,
               semt0, semt1, semg0, semg1, semw0, semw1):
    wid = lax.axis_index("s") * NUM_CORES + lax.axis_index("c")
    base = wid * B_PER_WORKER
    qb_base = wid * (B_PER_WORKER // 128)

    def start_t(g, t_v, semt):
        pltpu.async_copy(t_hbm.at[pl.ds(base + g * CHUNK, CHUNK)], t_v, semt)

    def wait_t(g, t_v, semt):
        pltpu.make_async_copy(
            t_hbm.at[pl.ds(base + g * CHUNK, CHUNK)], t_v, semt).wait()

    def compute_idx(t_v, idx_v):
        def body(r, carry):
            for c in range(GATHER_W // LANES):
                tv = t_v[pl.ds(r * GATHER_W + c * LANES, LANES)]
                ix = (tv * SCALE).astype(jnp.int32)
                ix = jnp.minimum(jnp.maximum(ix, 0), N_SEGMENTS - 1)
                idx_v[r, pl.ds(c * LANES, LANES)] = ix
            return carry

        lax.fori_loop(0, KG, body, 0, unroll=False)

    def fire_gathers(idx_v, rows_v, semg):
        for j in range(KG):
            pltpu.async_copy(
                amp_hbm.at[idx_v.at[j]],
                rows_v.at[pl.ds(j * GATHER_W, GATHER_W)], semg)

    def drain_gathers(idx_v, rows_v, semg):
        for j in range(KG):
            pltpu.make_async_copy(
                amp_hbm.at[idx_v.at[j]],
                rows_v.at[pl.ds(j * GATHER_W, GATHER_W)], semg).wait()

    HALF = QB * 1024
    iot = lax.iota(jnp.int32, LANES)
    base17 = iot * 17

    def transpose(rows_v, obuf, sbuf):
        def body(k, carry):
            q0 = k * LANES
            dynbase = ((q0 >> 7) << 10) + (q0 & 127)
            vs = [rows_v[q0 + u] for u in range(LANES)]
            for u in range(LANES):
                sbuf[pl.ds(u * 17, LANES)] = vs[u]
            cols = [plsc.load_gather(sbuf, [base17 + c]) for c in range(16)]
            for c in range(16):
                off_c = (c // 8) * HALF + (c % 8) * 128
                obuf[pl.ds(dynbase + off_c, LANES)] = cols[c]
            return carry

        lax.fori_loop(0, CHUNK // LANES, body, 0, unroll=False)

    def start_wb(g, obuf, semw):
        qo = (qb_base + g * QB) * 1024
        pltpu.async_copy(obuf.at[pl.ds(0, HALF)],
                         out_hbm.at[0, pl.ds(qo, HALF)], semw)
        pltpu.async_copy(obuf.at[pl.ds(HALF, HALF)],
                         out_hbm.at[1, pl.ds(qo, HALF)], semw)

    def wait_wb(g, obuf, semw):
        qo = (qb_base + g * QB) * 1024
        pltpu.make_async_copy(obuf.at[pl.ds(0, HALF)],
                              out_hbm.at[0, pl.ds(qo, HALF)], semw).wait()
        pltpu.make_async_copy(obuf.at[pl.ds(HALF, HALF)],
                              out_hbm.at[1, pl.ds(qo, HALF)], semw).wait()

    slot = [(t_v0, idx_v0, rows0, ob0, semt0, semg0, semw0),
            (t_v1, idx_v1, rows1, ob1, semt1, semg1, semw1)]

    start_t(0, t_v0, semt0)
    start_t(1, t_v1, semt1)
    wait_t(0, t_v0, semt0)
    compute_idx(t_v0, idx_v0)
    start_t(2, t_v0, semt0)
    fire_gathers(idx_v0, rows0, semg0)
    wait_t(1, t_v1, semt1)
    compute_idx(t_v1, idx_v1)
    start_t(3, t_v1, semt1)
    fire_gathers(idx_v1, rows1, semg1)
    drain_gathers(idx_v0, rows0, semg0)
    transpose(rows0, ob0, sbuf)
    start_wb(0, ob0, semw0)
    wait_t(2, t_v0, semt0)
    compute_idx(t_v0, idx_v0)
    start_t(4, t_v0, semt0)
    fire_gathers(idx_v0, rows0, semg0)
    drain_gathers(idx_v1, rows1, semg1)
    transpose(rows1, ob1, sbuf)
    start_wb(1, ob1, semw1)
    wait_t(3, t_v1, semt1)
    compute_idx(t_v1, idx_v1)
    start_t(5, t_v1, semt1)
    fire_gathers(idx_v1, rows1, semg1)
    drain_gathers(idx_v0, rows0, semg0)
    wait_wb(0, ob0, semw0)
    transpose(rows0, ob0, sbuf)
    start_wb(2, ob0, semw0)

    def step(s, carry):
        g0 = 2 * s
        g1 = g0 + 1
        wait_t(g0, t_v0, semt0)
        compute_idx(t_v0, idx_v0)
        start_t(g0 + 2, t_v0, semt0)
        fire_gathers(idx_v0, rows0, semg0)
        drain_gathers(idx_v1, rows1, semg1)
        wait_wb(g0 - 3, ob1, semw1)
        transpose(rows1, ob1, sbuf)
        start_wb(g0 - 1, ob1, semw1)
        wait_t(g1, t_v1, semt1)
        compute_idx(t_v1, idx_v1)
        start_t(g1 + 2, t_v1, semt1)
        fire_gathers(idx_v1, rows1, semg1)
        drain_gathers(idx_v0, rows0, semg0)
        wait_wb(g1 - 3, ob0, semw0)
        transpose(rows0, ob0, sbuf)
        start_wb(g0, ob0, semw0)
        return carry

    lax.fori_loop(2, N_CHUNKS // 2 - 1, step, 0, unroll=False)

    gA = N_CHUNKS - 2
    gB = N_CHUNKS - 1
    wait_t(gA, t_v0, semt0)
    compute_idx(t_v0, idx_v0)
    fire_gathers(idx_v0, rows0, semg0)
    drain_gathers(idx_v1, rows1, semg1)
    wait_wb(gA - 3, ob1, semw1)
    transpose(rows1, ob1, sbuf)
    start_wb(gA - 1, ob1, semw1)

    wait_t(gB, t_v1, semt1)
    compute_idx(t_v1, idx_v1)
    fire_gathers(idx_v1, rows1, semg1)
    drain_gathers(idx_v0, rows0, semg0)
    wait_wb(gB - 3, ob0, semw0)
    transpose(rows0, ob0, sbuf)
    start_wb(gA, ob0, semw0)

    drain_gathers(idx_v1, rows1, semg1)
    wait_wb(gA - 1, ob1, semw1)
    transpose(rows1, ob1, sbuf)
    start_wb(gB, ob1, semw1)
    wait_wb(gA, ob0, semw0)
    wait_wb(gB, ob1, semw1)


@jax.jit
def kernel(t, amplitudes):
    mesh = plsc.VectorSubcoreMesh(core_axis_name="c", subcore_axis_name="s")
    fmt = functools.partial(
        pl.kernel,
        mesh=mesh,
        out_type=jax.ShapeDtypeStruct((N_ROWS_PAD, 128), jnp.float32),
        scratch_types=[
            pltpu.VMEM((16, 128), jnp.float32),
            pltpu.VMEM((16, 128), jnp.float32),
            pltpu.VMEM((16, 128), jnp.float32),
            pltpu.VMEM((16, 128), jnp.float32),
            pltpu.VMEM((16 * 17,), jnp.float32),
            pltpu.SemaphoreType.DMA,
            pltpu.SemaphoreType.DMA,
            pltpu.SemaphoreType.DMA,
            pltpu.SemaphoreType.DMA,
        ],
        compiler_params=pltpu.CompilerParams(
            use_tc_tiling_on_sc=True, needs_layout_passes=False,
            disable_bounds_checks=True),
    )(_sc_format)
    amp_lin = fmt(amplitudes.T).reshape(N_ROWS_PAD * 8, 16)
    run = functools.partial(
        pl.kernel,
        mesh=mesh,
        out_type=jax.ShapeDtypeStruct((2, N_QB * 1024), jnp.float32),
        scratch_types=[
            pltpu.VMEM((CHUNK,), jnp.float32),
            pltpu.VMEM((CHUNK,), jnp.float32),
            pltpu.VMEM((KG, GATHER_W), jnp.int32),
            pltpu.VMEM((KG, GATHER_W), jnp.int32),
            pltpu.VMEM((CHUNK, N_CHANNELS), jnp.float32),
            pltpu.VMEM((CHUNK, N_CHANNELS), jnp.float32),
            pltpu.VMEM((2 * QB * 1024,), jnp.float32),
            pltpu.VMEM((2 * QB * 1024,), jnp.float32),
            pltpu.VMEM((16 * 17,), jnp.float32),
            pltpu.SemaphoreType.DMA,
            pltpu.SemaphoreType.DMA,
            pltpu.SemaphoreType.DMA,
            pltpu.SemaphoreType.DMA,
            pltpu.SemaphoreType.DMA,
            pltpu.SemaphoreType.DMA,
        ],
        compiler_params=pltpu.CompilerParams(
            use_tc_tiling_on_sc=False, needs_layout_passes=False,
            disable_bounds_checks=True),
    )(_sc_gather)
    out4 = run(t, amp_lin).reshape(2, N_QB, 8, 128)
    return out4.transpose(1, 3, 0, 2).reshape(N_TIMES, N_CHANNELS)
